# Initial kernel scaffold; baseline (speedup 1.0000x reference)
import jax
import jax.numpy as jnp
from jax.experimental import pallas as pl


def kernel(poi_embs, edge_embs, inc_index, vals_p2e, vals_e2p, W_poi, W_edge, W_fus):
    raise NotImplementedError("write your pallas kernel here")


# speedup vs baseline: 4.0518x; 4.0518x over previous
import functools

import jax
import jax.numpy as jnp
from jax import lax
from jax.experimental import pallas as pl
from jax.experimental.pallas import tpu as pltpu
from jax.experimental.pallas import tpu_sc as plsc

N = 10000
NNZ = 320000
D = 128
NCORES = 2
NTILES = 16
CHUNK = 80
NCHUNKS = NNZ // CHUNK
CPW = NCHUNKS // (NCORES * NTILES)
NPAD = 10240
RPT = NPAD // NTILES
BLK = 2000


def _spmm_partials(x, src, dst, vals):
    mesh = plsc.VectorSubcoreMesh(core_axis_name="c", subcore_axis_name="s")

    @functools.partial(
        pl.kernel,
        out_type=jax.ShapeDtypeStruct((NCORES, NPAD, D), jnp.float32),
        mesh=mesh,
        scratch_types=[
            pltpu.VMEM((CHUNK,), jnp.int32),
            pltpu.VMEM((CHUNK,), jnp.int32),
            pltpu.VMEM((CHUNK,), jnp.float32),
            pltpu.VMEM((CHUNK, D), jnp.float32),
            pltpu.VMEM_SHARED((NPAD, D), jnp.float32),
            pltpu.SemaphoreType.DMA,
        ],
    )
    def sc_kernel(x_hbm, src_hbm, dst_hbm, vals_hbm, out_hbm,
                  src_v, dst_v, vals_v, rows_v, acc, sem):
        cid = lax.axis_index("c")
        sid = lax.axis_index("s")
        wid = sid * NCORES + cid

        def zero_row(k, carry):
            for j in range(D // 16):
                rows_v[k, pl.ds(16 * j, 16)] = jnp.zeros((16,), jnp.float32)
            return carry
        lax.fori_loop(0, CHUNK, zero_row, 0)
        base = sid * RPT
        nfull = RPT // CHUNK
        for r in range(nfull):
            pltpu.sync_copy(rows_v, acc.at[pl.ds(base + r * CHUNK, CHUNK)])
        rem = RPT - nfull * CHUNK
        if rem:
            pltpu.sync_copy(rows_v.at[pl.ds(0, rem)],
                            acc.at[pl.ds(base + nfull * CHUNK, rem)])
        plsc.subcore_barrier()

        def chunk_body(i, carry):
            off = (wid + NCORES * NTILES * i) * CHUNK
            pltpu.sync_copy(src_hbm.at[pl.ds(off, CHUNK)], src_v)
            pltpu.sync_copy(dst_hbm.at[pl.ds(off, CHUNK)], dst_v)
            pltpu.sync_copy(vals_hbm.at[pl.ds(off, CHUNK)], vals_v)
            pltpu.async_copy(x_hbm.at[src_v], rows_v, sem).wait()

            def scale(g, c2):
                v16 = vals_v[pl.ds(16 * g, 16)]
                base_r = 16 * g
                for r in range(16):
                    v = v16[r]
                    for j in range(D // 16):
                        sl = pl.ds(16 * j, 16)
                        rows_v[base_r + r, sl] = rows_v[base_r + r, sl] * v
                return c2
            lax.fori_loop(0, CHUNK // 16, scale, 0)
            pltpu.sync_copy(rows_v, acc.at[dst_v], add=True)
            return carry
        lax.fori_loop(0, CPW, chunk_body, 0)

        plsc.subcore_barrier()
        pltpu.sync_copy(acc.at[pl.ds(base, RPT)],
                        out_hbm.at[cid, pl.ds(base, RPT)])

    return sc_kernel(x, src, dst, vals)


def _dgt(x, w):
    return lax.dot_general(x, w, (((1,), (1,)), ((), ())),
                           preferred_element_type=jnp.float32)


_GRID = (N // BLK,)
_row = pl.BlockSpec((BLK, D), lambda i: (i, 0))
_pair = pl.BlockSpec((NCORES, BLK, D), lambda i: (0, i, 0))
_wfull = pl.BlockSpec((D, D), lambda i: (0, 0))
_OUT_ROW = jax.ShapeDtypeStruct((N, D), jnp.float32)


def _tc_weight(wf2, wedge):
    def body(a_ref, b_ref, o_ref):
        o_ref[...] = lax.dot_general(a_ref[...], b_ref[...],
                                     (((1,), (0,)), ((), ())),
                                     preferred_element_type=jnp.float32)
    return pl.pallas_call(
        body, out_shape=jax.ShapeDtypeStruct((D, D), jnp.float32))(wf2, wedge)


def _tc_poi1(p, w):
    def body(x_ref, w_ref, o_ref):
        o_ref[...] = _dgt(x_ref[...], w_ref[...])
    return pl.pallas_call(
        body, grid=_GRID,
        in_specs=[_row, _wfull], out_specs=_row, out_shape=_OUT_ROW)(p, w)


def _tc_poi2(p, prop, w):
    def body(x_ref, pp_ref, w_ref, o_ref):
        xs = x_ref[...] + pp_ref[0] + pp_ref[1]
        o_ref[...] = _dgt(xs, w_ref[...])
    return pl.pallas_call(
        body, grid=_GRID,
        in_specs=[_row, _pair, _wfull], out_specs=_row,
        out_shape=_OUT_ROW)(p, prop, w)


def _tc_fuse1(m, e, wf1, c):
    def body(m_ref, e_ref, w1_ref, c_ref, o_ref):
        msum = m_ref[0] + m_ref[1]
        o_ref[...] = _dgt(msum, w1_ref[...]) + _dgt(e_ref[...], c_ref[...])
    return pl.pallas_call(
        body, grid=_GRID,
        in_specs=[_pair, _row, _wfull, _wfull], out_specs=_row,
        out_shape=_OUT_ROW)(m, e, wf1, c)


def _tc_fuse2(m2, e0, f1, wf1, c):
    def body(m_ref, e_ref, f1_ref, w1_ref, c_ref, f2_ref, eo_ref):
        msum = m_ref[0] + m_ref[1]
        e1 = e_ref[...] + f1_ref[...]
        f2 = _dgt(msum, w1_ref[...]) + _dgt(e1, c_ref[...])
        f2_ref[...] = f2
        eo_ref[...] = e_ref[...] + (2.0 * f1_ref[...] + f2) * (1.0 / 3.0)
    return pl.pallas_call(
        body, grid=_GRID,
        in_specs=[_pair, _row, _row, _wfull, _wfull],
        out_specs=[_row, _row],
        out_shape=[_OUT_ROW, _OUT_ROW])(m2, e0, f1, wf1, c)


def _tc_poi_out(p0, prop1, prop2):
    def body(p_ref, p1_ref, p2_ref, o_ref):
        s1 = p1_ref[0] + p1_ref[1]
        s2 = p2_ref[0] + p2_ref[1]
        o_ref[...] = p_ref[...] + (2.0 * s1 + s2) * (1.0 / 3.0)
    return pl.pallas_call(
        body, grid=_GRID,
        in_specs=[_row, _pair, _pair], out_specs=_row,
        out_shape=_OUT_ROW)(p0, prop1, prop2)


def kernel(poi_embs, edge_embs, inc_index, vals_p2e, vals_e2p,
           W_poi, W_edge, W_fus):
    e_ids = inc_index[0]
    p_ids = inc_index[1]
    wf1 = W_fus[:, :D]
    wf2 = W_fus[:, D:]
    c = _tc_weight(wf2, W_edge)

    xp1 = _tc_poi1(poi_embs, W_poi)
    m1 = _spmm_partials(xp1, p_ids, e_ids, vals_p2e)
    f1 = _tc_fuse1(m1, edge_embs, wf1, c)
    prop1 = _spmm_partials(f1, e_ids, p_ids, vals_e2p)

    xp2 = _tc_poi2(poi_embs, prop1, W_poi)
    m2 = _spmm_partials(xp2, p_ids, e_ids, vals_p2e)
    f2, edge_out = _tc_fuse2(m2, edge_embs, f1, wf1, c)
    prop2 = _spmm_partials(f2, e_ids, p_ids, vals_e2p)

    poi_out = _tc_poi_out(poi_embs, prop1, prop2)
    return (poi_out, edge_out)
